# Initial kernel scaffold; baseline (speedup 1.0000x reference)
import jax
import jax.numpy as jnp
from jax.experimental import pallas as pl


def kernel(hm, reg, dep, dim, rot, hm_target, reg_mask, ind, reg_target, dep_target, dim_target, rot_target):
    raise NotImplementedError("write your pallas kernel here")


# speedup vs baseline: 2.2485x; 2.2485x over previous
import functools

import jax
import jax.numpy as jnp
from jax import lax
from jax.experimental import pallas as pl
from jax.experimental.pallas import tpu as pltpu
from jax.experimental.pallas import tpu_sc as plsc

B = 16
K = 128
HW = 128 * 128
NTILES = 32
PAIRS_PER_TILE = (B * K) // NTILES
CHANS = (2, 1, 3, 8)
NROWS = sum(CHANS)


def _dep_transform(x):
    s = 1.0 / (1.0 + jnp.exp(-x))
    return 1.0 / (s + 1e-6) - 1.0


def _sc_gather_losses(ind_flat, mask_flat, regf, depf, dimf, rotf,
                      regt, dept, dimt, rott):
    mesh = plsc.VectorSubcoreMesh(core_axis_name="c", subcore_axis_name="s")

    @functools.partial(
        pl.kernel,
        mesh=mesh,
        out_type=jax.ShapeDtypeStruct((NTILES, 4 * 16), jnp.float32),
        scratch_types=[
            pltpu.VMEM((PAIRS_PER_TILE,), jnp.int32),
            pltpu.VMEM((PAIRS_PER_TILE,), jnp.int32),
            pltpu.VMEM((2 * NROWS, PAIRS_PER_TILE), jnp.int32),
            pltpu.VMEM((2 * NROWS, PAIRS_PER_TILE), jnp.float32),
            pltpu.VMEM((4 * 16,), jnp.float32),
            pltpu.SemaphoreType.DMA,
        ],
    )
    def sc_kernel(ind_hbm, mask_hbm, reg_hbm, dep_hbm, dim_hbm, rot_hbm,
                  regt_hbm, dept_hbm, dimt_hbm, rott_hbm, out_hbm,
                  ind_v, mask_v, idx_v, vals_v, outb_v, sem):
        wid = lax.axis_index("c") * 16 + lax.axis_index("s")
        b = wid // 2
        base = wid * PAIRS_PER_TILE

        pltpu.sync_copy(ind_hbm.at[pl.ds(base, PAIRS_PER_TILE)], ind_v)
        pltpu.sync_copy(mask_hbm.at[pl.ds(base, PAIRS_PER_TILE)], mask_v)

        iota = lax.iota(jnp.int32, 16)
        row = 0
        for c_total in CHANS:
            for c in range(c_total):
                foff = (b * c_total + c) * HW
                for j in range(PAIRS_PER_TILE // 16):
                    sl = pl.ds(16 * j, 16)
                    idx_v[row, sl] = ind_v[sl] + foff
                    idx_v[NROWS + row, sl] = (base + 16 * j + iota) * c_total + c
                row += 1

        handles = []
        row = 0
        for c_total, src, tsrc in zip(CHANS,
                                      (reg_hbm, dep_hbm, dim_hbm, rot_hbm),
                                      (regt_hbm, dept_hbm, dimt_hbm, rott_hbm)):
            for _ in range(c_total):
                handles.append(
                    pltpu.async_copy(src.at[idx_v.at[row]], vals_v.at[row], sem))
                handles.append(
                    pltpu.async_copy(tsrc.at[idx_v.at[NROWS + row]],
                                     vals_v.at[NROWS + row], sem))
                row += 1
        for h in handles:
            h.wait()

        row = 0
        for ai, c_total in enumerate(CHANS):
            acc = jnp.zeros((16,), jnp.float32)
            for _ in range(c_total):
                for j in range(PAIRS_PER_TILE // 16):
                    sl = pl.ds(16 * j, 16)
                    pred = vals_v[row, sl]
                    if ai == 1:
                        pred = _dep_transform(pred)
                    m = mask_v[sl].astype(jnp.float32)
                    tv = vals_v[NROWS + row, sl]
                    acc = acc + jnp.abs(pred * m - tv * m)
                row += 1
            outb_v[pl.ds(16 * ai, 16)] = acc
        pltpu.sync_copy(outb_v, out_hbm.at[wid])

    return sc_kernel(ind_flat, mask_flat, regf, depf, dimf, rotf,
                     regt, dept, dimt, rott)


_NB = 12
_BR = (B * 3 * 128) // _NB


def _tc_body(hm_ref, t_ref, p_ref, o_tot, o_hm, o_off, o_dep, o_dim, o_rot,
             acc_ref):
    i = pl.program_id(0)

    @pl.when(i == 0)
    def _init():
        acc_ref[...] = jnp.zeros_like(acc_ref)

    x = hm_ref[...]
    t = t_ref[...]
    s = jnp.clip(1.0 / (1.0 + jnp.exp(-x)), 1e-4, 1.0 - 1e-4)
    d = s - t
    acc_ref[...] += jnp.sum(d * d, axis=0, keepdims=True)

    @pl.when(i == _NB - 1)
    def _fin():
        p = p_ref[...]
        hm_l = jnp.sum(acc_ref[...]) / (B * 3.0 * HW)
        off_l = jnp.sum(p[:, 0:16]) / (B * K * 2.0)
        dep_l = jnp.sum(p[:, 16:32]) / (B * K * 1.0)
        dim_l = jnp.sum(p[:, 32:48]) / (B * K * 3.0)
        rot_l = jnp.sum(p[:, 48:64]) / (B * K * 8.0)
        o_hm[0, 0] = hm_l
        o_off[0, 0] = off_l
        o_dep[0, 0] = dep_l
        o_dim[0, 0] = dim_l
        o_rot[0, 0] = rot_l
        o_tot[0, 0] = hm_l + off_l + dep_l + dim_l + rot_l


def _tc_combine(hm2, hmt2, partials):
    scalar = jax.ShapeDtypeStruct((1, 1), jnp.float32)
    return pl.pallas_call(
        _tc_body,
        grid=(_NB,),
        in_specs=[
            pl.BlockSpec((_BR, 128), lambda i: (i, 0)),
            pl.BlockSpec((_BR, 128), lambda i: (i, 0)),
            pl.BlockSpec((NTILES, 64), lambda i: (0, 0)),
        ],
        out_specs=[pl.BlockSpec((1, 1), lambda i: (0, 0),
                                memory_space=pltpu.SMEM)] * 6,
        out_shape=[scalar] * 6,
        scratch_shapes=[pltpu.VMEM((1, 128), jnp.float32)],
    )(hm2, hmt2, partials)


def kernel(hm, reg, dep, dim, rot, hm_target, reg_mask, ind, reg_target,
           dep_target, dim_target, rot_target):
    ind_flat = ind.astype(jnp.int32).reshape(-1)
    mask_flat = reg_mask.astype(jnp.int32).reshape(-1)
    partials = _sc_gather_losses(
        ind_flat, mask_flat,
        reg.reshape(-1), dep.reshape(-1), dim.reshape(-1), rot.reshape(-1),
        reg_target.reshape(-1), dep_target.reshape(-1),
        dim_target.reshape(-1), rot_target.reshape(-1))
    outs = _tc_combine(hm.reshape(B * 3 * 128, 128),
                       hm_target.reshape(B * 3 * 128, 128), partials)
    tot, hm_l, off_l, dep_l, dim_l, rot_l = [o.reshape(()) for o in outs]
    return (tot, hm_l, off_l, dep_l, dim_l, rot_l)


# speedup vs baseline: 2.8540x; 1.2693x over previous
import functools

import jax
import jax.numpy as jnp
from jax import lax
from jax.experimental import pallas as pl
from jax.experimental.pallas import tpu as pltpu
from jax.experimental.pallas import tpu_sc as plsc

B = 16
K = 128
HW = 128 * 128
NTILES = 32
PAIRS_PER_TILE = (B * K) // NTILES
CHANS = (2, 1, 3, 8)
NROWS = sum(CHANS)


def _dep_transform(x):
    s = 1.0 / (1.0 + jnp.exp(-x))
    return 1.0 / (s + 1e-6) - 1.0


def _sc_gather_losses(ind_flat, mask_flat, regf, depf, dimf, rotf, tcat):
    mesh = plsc.VectorSubcoreMesh(core_axis_name="c", subcore_axis_name="s")

    @functools.partial(
        pl.kernel,
        mesh=mesh,
        out_type=jax.ShapeDtypeStruct((NTILES * 64,), jnp.float32),
        scratch_types=[
            pltpu.VMEM((PAIRS_PER_TILE,), jnp.int32),
            pltpu.VMEM((PAIRS_PER_TILE,), jnp.int32),
            pltpu.VMEM((2 * NROWS, PAIRS_PER_TILE), jnp.int32),
            pltpu.VMEM((2 * NROWS, PAIRS_PER_TILE), jnp.float32),
            pltpu.VMEM((4 * 16,), jnp.float32),
            pltpu.SemaphoreType.DMA,
        ],
    )
    def sc_kernel(ind_hbm, mask_hbm, reg_hbm, dep_hbm, dim_hbm, rot_hbm,
                  tcat_hbm, out_hbm, ind_v, mask_v, idx_v, vals_v, outb_v,
                  sem):
        wid = lax.axis_index("c") * 16 + lax.axis_index("s")
        b = wid // 2
        base = wid * PAIRS_PER_TILE

        pltpu.sync_copy(ind_hbm.at[pl.ds(base, PAIRS_PER_TILE)], ind_v)
        pltpu.sync_copy(mask_hbm.at[pl.ds(base, PAIRS_PER_TILE)], mask_v)

        iota = lax.iota(jnp.int32, 16)
        row = 0
        for c_total in CHANS:
            for c in range(c_total):
                foff = (b * c_total + c) * HW
                for j in range(PAIRS_PER_TILE // 16):
                    sl = pl.ds(16 * j, 16)
                    idx_v[row, sl] = ind_v[sl] + foff
                    idx_v[NROWS + row, sl] = (base + 16 * j + iota) * NROWS + row
                row += 1

        handles = []
        row = 0
        for c_total, src in zip(CHANS, (reg_hbm, dep_hbm, dim_hbm, rot_hbm)):
            for _ in range(c_total):
                handles.append(
                    pltpu.async_copy(src.at[idx_v.at[row]], vals_v.at[row], sem))
                handles.append(
                    pltpu.async_copy(tcat_hbm.at[idx_v.at[NROWS + row]],
                                     vals_v.at[NROWS + row], sem))
                row += 1
        for h in handles:
            h.wait()

        row = 0
        for ai, c_total in enumerate(CHANS):
            acc = jnp.zeros((16,), jnp.float32)
            for _ in range(c_total):
                for j in range(PAIRS_PER_TILE // 16):
                    sl = pl.ds(16 * j, 16)
                    pred = vals_v[row, sl]
                    if ai == 1:
                        pred = _dep_transform(pred)
                    m = mask_v[sl].astype(jnp.float32)
                    tv = vals_v[NROWS + row, sl]
                    acc = acc + jnp.abs(pred * m - tv * m)
                row += 1
            outb_v[pl.ds(16 * ai, 16)] = acc
        pltpu.sync_copy(outb_v, out_hbm.at[pl.ds(base, 64)])

    return sc_kernel(ind_flat, mask_flat, regf, depf, dimf, rotf, tcat)


_NB = 12
_BR = (B * 3 * 128) // _NB


def _mse_body(hm_ref, t_ref, o_ref):
    i = pl.program_id(0)
    x = hm_ref[...]
    t = t_ref[...]
    s = jnp.clip(1.0 / (1.0 + jnp.exp(-x)), 1e-4, 1.0 - 1e-4)
    d = s - t
    ps = jnp.sum(d * d, axis=0, keepdims=True)

    @pl.when(i == 0)
    def _init():
        o_ref[...] = ps

    @pl.when(i > 0)
    def _acc():
        o_ref[...] += ps


def _tc_mse(hm2, hmt2):
    return pl.pallas_call(
        _mse_body,
        grid=(_NB,),
        in_specs=[
            pl.BlockSpec((_BR, 128), lambda i: (i, 0)),
            pl.BlockSpec((_BR, 128), lambda i: (i, 0)),
        ],
        out_specs=pl.BlockSpec((1, 128), lambda i: (0, 0)),
        out_shape=jax.ShapeDtypeStruct((1, 128), jnp.float32),
    )(hm2, hmt2)


def _combine_body(mse_ref, p_ref, o_tot, o_hm, o_off, o_dep, o_dim, o_rot):
    p = p_ref[...]
    hm_l = jnp.sum(mse_ref[...]) / (B * 3.0 * HW)
    ls = []
    for l in range(4):
        ls.append((jnp.sum(p[:, 16 * l:16 * l + 16]) +
                   jnp.sum(p[:, 64 + 16 * l:80 + 16 * l])))
    off_l = ls[0] / (B * K * 2.0)
    dep_l = ls[1] / (B * K * 1.0)
    dim_l = ls[2] / (B * K * 3.0)
    rot_l = ls[3] / (B * K * 8.0)
    o_hm[0, 0] = hm_l
    o_off[0, 0] = off_l
    o_dep[0, 0] = dep_l
    o_dim[0, 0] = dim_l
    o_rot[0, 0] = rot_l
    o_tot[0, 0] = hm_l + off_l + dep_l + dim_l + rot_l


def _tc_combine(mse, partials2d):
    scalar = jax.ShapeDtypeStruct((1, 1), jnp.float32)
    return pl.pallas_call(
        _combine_body,
        out_specs=[pl.BlockSpec(memory_space=pltpu.SMEM)] * 6,
        out_shape=[scalar] * 6,
    )(mse, partials2d)


def kernel(hm, reg, dep, dim, rot, hm_target, reg_mask, ind, reg_target,
           dep_target, dim_target, rot_target):
    ind_flat = ind.astype(jnp.int32).reshape(-1)
    mask_flat = reg_mask.astype(jnp.int32).reshape(-1)
    tcat = jnp.concatenate(
        [reg_target, dep_target, dim_target, rot_target], axis=2).reshape(-1)
    partials = _sc_gather_losses(
        ind_flat, mask_flat,
        reg.reshape(-1), dep.reshape(-1), dim.reshape(-1), rot.reshape(-1),
        tcat)
    mse = _tc_mse(hm.reshape(B * 3 * 128, 128),
                  hm_target.reshape(B * 3 * 128, 128))
    outs = _tc_combine(mse, partials.reshape(16, 128))
    tot, hm_l, off_l, dep_l, dim_l, rot_l = [o.reshape(()) for o in outs]
    return (tot, hm_l, off_l, dep_l, dim_l, rot_l)
